# baseline (speedup 1.0000x reference)
import functools

import jax
import jax.numpy as jnp
from jax import lax
from jax.experimental import pallas as pl
from jax.experimental.pallas import tpu as pltpu
from jax.experimental.pallas import tpu_sc as plsc

_P = 100000
_NPAT = 100000
_TR = 4000
_NTR = _NPAT // _TR
_E = 64
_OC = 96

_B, _C, _H, _W = 4, 96, 224, 224
_CB = 16
_NC = _C // _CB

_NPIX = _H * _W
_NW = 32
_BPW = _NPIX // _NW
_CH = 784
_NCH = _BPW // _CH
_NB = 2
_IDXW = _BPW + 112

_KP = 4
_NK = _H // (2 * _KP)


def _hash_body(x_ref, acc_ref):
    c = pl.program_id(0)
    q = (x_ref[...] * 100.0).astype(jnp.int32)
    left = jnp.concatenate([q[:, :, :1], q[:, :, :-1]], axis=2)
    right = jnp.concatenate([q[:, :, 1:], q[:, :, -1:]], axis=2)
    hs = left + q + right
    up = jnp.concatenate([hs[:, :1, :], hs[:, :-1, :]], axis=1)
    down = jnp.concatenate([hs[:, 1:, :], hs[:, -1:, :]], axis=1)
    s = up + hs + down
    sa = jnp.abs(s)
    q0 = (sa.astype(jnp.float32) * (1.0 / _P)).astype(jnp.int32)
    r = sa - q0 * _P
    r = jnp.where(r < 0, r + _P, r)
    r = jnp.where(r >= _P, r - _P, r)
    part = jnp.sum(r, axis=0)

    @pl.when(c == 0)
    def _init():
        acc_ref[...] = part

    @pl.when(c != 0)
    def _acc():
        acc_ref[...] += part


def _hash_sum(x_r, b):
    return pl.pallas_call(
        _hash_body,
        grid=(_NC,),
        in_specs=[pl.BlockSpec((_CB, _H, _W), lambda c, b=b: (b * _NC + c, 0, 0))],
        out_specs=pl.BlockSpec((_H, _W), lambda c: (0, 0)),
        out_shape=jax.ShapeDtypeStruct((_H, _W), jnp.int32),
    )(x_r)


def _gather_body(table_hbm, idx_hbm, out_hbm, idx_v, idx_p, rows, gsems, ssems):
    wid = lax.axis_index("s") * 2 + lax.axis_index("c")
    base = wid * _BPW
    start = base - 112 * lax.bitwise_and(wid, 1)
    pltpu.sync_copy(idx_hbm.at[pl.ds(start, _IDXW)], idx_v)

    lanes = lax.iota(jnp.int32, 16)

    def permute(g, carry):
        j = g * 16
        s = lanes + (j + base)
        t = lax.bitwise_and(s, 1)
        t7 = lax.shift_right_logical(s, 6)
        k = ((t7.astype(jnp.float32) + 0.5) * (1.0 / 7.0)).astype(jnp.int32)
        r = lax.shift_right_logical(s, 1) - 224 * k
        p_local = (2 * k + t) * 224 + r - start
        idx_p[pl.ds(j, 16)] = plsc.load_gather(idx_v, [p_local])
        return carry

    lax.fori_loop(0, _BPW // 16, permute, 0)

    def start_gather(i, b):
        return pltpu.async_copy(
            table_hbm.at[idx_p.at[pl.ds(i * _CH, _CH)]], rows[b], gsems[b])

    gath = [start_gather(j, j) for j in range(min(_NB, _NCH))]
    scat = [None] * _NB
    for i in range(_NCH):
        b = i % _NB
        gath[b].wait()
        scat[b] = pltpu.async_copy(
            rows[b], out_hbm.at[pl.ds(base + i * _CH, _CH)], ssems[b])
        if i + _NB < _NCH:
            scat[b].wait()
            gath[b] = start_gather(i + _NB, b)
    for i in range(max(0, _NCH - _NB), _NCH):
        scat[i % _NB].wait()


@functools.cache
def _make_gather():
    return functools.partial(
        pl.kernel,
        mesh=plsc.VectorSubcoreMesh(core_axis_name="c", subcore_axis_name="s"),
        compiler_params=pltpu.CompilerParams(
            use_tc_tiling_on_sc=False, needs_layout_passes=False),
        out_type=jax.ShapeDtypeStruct((_NPIX, _E), jnp.float32),
        scratch_types=[
            pltpu.VMEM((_IDXW,), jnp.int32),
            pltpu.VMEM((_BPW,), jnp.int32),
            [pltpu.VMEM((_CH, _E), jnp.float32) for _ in range(_NB)],
            [pltpu.SemaphoreType.DMA for _ in range(_NB)],
            [pltpu.SemaphoreType.DMA for _ in range(_NB)],
        ],
    )(_gather_body)


def _compact_body(t_ref, out_ref):
    v = t_ref[...].reshape(_TR // 2, 2, _E)
    out_ref[:, :_E] = v[:, 0, :]
    out_ref[:, _E:] = v[:, 1, :]


def _compact_table(table):
    return pl.pallas_call(
        _compact_body,
        grid=(_NTR,),
        in_specs=[pl.BlockSpec((_TR, _E), lambda i: (i, 0))],
        out_specs=pl.BlockSpec((_TR // 2, 2 * _E), lambda i: (i, 0)),
        out_shape=jax.ShapeDtypeStruct((_NPAT // 2, 2 * _E), jnp.float32),
    )(table)


def _proj_body_first(emb_ref, w2_ref, b_ref, out_ref):
    w2 = w2_ref[...]
    bias = b_ref[...]
    for i in range(_KP):
        e2 = emb_ref[pl.ds(i * _W, _W), :]
        r2 = lax.dot_general(w2, e2, (((1,), (1,)), ((), ())),
                             preferred_element_type=jnp.float32)
        out_ref[0, :, 2 * i, :] = r2[:_OC] + bias
        out_ref[0, :, 2 * i + 1, :] = r2[_OC:] + bias


def _proj_body(prev_ref, emb_ref, w2_ref, b_ref, out_ref):
    del prev_ref
    _proj_body_first(emb_ref, w2_ref, b_ref, out_ref)


def _project(prev, emb_b, w2, bias2d, b):
    emb2 = emb_b.reshape(_NPIX // 2, 2 * _E)
    common = dict(
        grid=(_NK,),
        out_specs=pl.BlockSpec((1, _OC, 2 * _KP, _W), lambda k, b=b: (b, 0, k, 0)),
        out_shape=jax.ShapeDtypeStruct((_B, _OC, _H, _W), jnp.float32),
    )
    emb_spec = pl.BlockSpec((_KP * _W, 2 * _E), lambda k: (k, 0))
    w_spec = pl.BlockSpec((2 * _OC, 2 * _E), lambda k: (0, 0))
    b_spec = pl.BlockSpec((_OC, 1), lambda k: (0, 0))
    if prev is None:
        return pl.pallas_call(
            _proj_body_first,
            in_specs=[emb_spec, w_spec, b_spec],
            **common,
        )(emb2, w2, bias2d)
    return pl.pallas_call(
        _proj_body,
        in_specs=[pl.BlockSpec(memory_space=pl.ANY),
                  emb_spec, w_spec, b_spec],
        input_output_aliases={0: 0},
        **common,
    )(prev, emb2, w2, bias2d)


def kernel(x, embedding_table, proj_weight, proj_bias):
    x_r = x.reshape(_B * _C, _H, _W)
    bias2d = proj_bias.reshape(_OC, 1)
    z = jnp.zeros((_OC, _E), proj_weight.dtype)
    w2 = jnp.concatenate(
        [jnp.concatenate([proj_weight, z], axis=1),
         jnp.concatenate([z, proj_weight], axis=1)], axis=0)
    gather = _make_gather()
    table_lin = _compact_table(embedding_table).reshape(_NPAT, _E)
    out = None
    for b in range(_B):
        acc = _hash_sum(x_r, b)
        idx = (acc.astype(jnp.float32) / jnp.float32(_C)).astype(jnp.int32)
        emb_b = gather(table_lin, idx.reshape(_NPIX))
        out = _project(out, emb_b, w2, bias2d, b)
    return out


# speedup vs baseline: 1.0928x; 1.0928x over previous
import functools

import jax
import jax.numpy as jnp
from jax import lax
from jax.experimental import pallas as pl
from jax.experimental.pallas import tpu as pltpu
from jax.experimental.pallas import tpu_sc as plsc

_P = 100000
_E = 64
_OC = 96

_B, _C, _H, _W = 4, 96, 224, 224
_CB = 16
_NC = _C // _CB

_NPIX = _H * _W
_NW = 32
_BPW = _NPIX // _NW
_CH = 784
_NCH = _BPW // _CH
_NB = 2
_IDXW = _BPW + 112

_KP = 4
_NK = _H // (2 * _KP)


def _hash_body(x_ref, acc_ref):
    c = pl.program_id(0)
    q = (x_ref[...] * 100.0).astype(jnp.int32)
    left = jnp.concatenate([q[:, :, :1], q[:, :, :-1]], axis=2)
    right = jnp.concatenate([q[:, :, 1:], q[:, :, -1:]], axis=2)
    hs = left + q + right
    up = jnp.concatenate([hs[:, :1, :], hs[:, :-1, :]], axis=1)
    down = jnp.concatenate([hs[:, 1:, :], hs[:, -1:, :]], axis=1)
    s = up + hs + down
    sa = jnp.abs(s)
    q0 = (sa.astype(jnp.float32) * (1.0 / _P)).astype(jnp.int32)
    r = sa - q0 * _P
    r = jnp.where(r < 0, r + _P, r)
    r = jnp.where(r >= _P, r - _P, r)
    part = jnp.sum(r, axis=0)

    @pl.when(c == 0)
    def _init():
        acc_ref[...] = part

    @pl.when(c != 0)
    def _acc():
        acc_ref[...] += part


def _hash_sum(x_r, b):
    return pl.pallas_call(
        _hash_body,
        grid=(_NC,),
        in_specs=[pl.BlockSpec((_CB, _H, _W), lambda c, b=b: (b * _NC + c, 0, 0))],
        out_specs=pl.BlockSpec((_H, _W), lambda c: (0, 0)),
        out_shape=jax.ShapeDtypeStruct((_H, _W), jnp.int32),
    )(x_r)


def _gather_body(table_hbm, idx_hbm, out_hbm, idx_v, idx_p, rows, gsems, ssems):
    wid = lax.axis_index("s") * 2 + lax.axis_index("c")
    base = wid * _BPW
    start = base - 112 * lax.bitwise_and(wid, 1)
    pltpu.sync_copy(idx_hbm.at[pl.ds(start, _IDXW)], idx_v)

    lanes = lax.iota(jnp.int32, 16)

    def permute(g, carry):
        j = g * 16
        s = lanes + (j + base)
        t = lax.bitwise_and(s, 1)
        t7 = lax.shift_right_logical(s, 6)
        k = ((t7.astype(jnp.float32) + 0.5) * (1.0 / 7.0)).astype(jnp.int32)
        r = lax.shift_right_logical(s, 1) - 224 * k
        p_local = (2 * k + t) * 224 + r - start
        idx_p[pl.ds(j, 16)] = plsc.load_gather(idx_v, [p_local])
        return carry

    lax.fori_loop(0, _BPW // 16, permute, 0)

    def start_gather(i, b):
        return pltpu.async_copy(
            table_hbm.at[idx_p.at[pl.ds(i * _CH, _CH)]], rows[b], gsems[b])

    gath = [start_gather(j, j) for j in range(min(_NB, _NCH))]
    scat = [None] * _NB
    for i in range(_NCH):
        b = i % _NB
        gath[b].wait()
        scat[b] = pltpu.async_copy(
            rows[b], out_hbm.at[pl.ds(base + i * _CH, _CH)], ssems[b])
        if i + _NB < _NCH:
            scat[b].wait()
            gath[b] = start_gather(i + _NB, b)
    for i in range(max(0, _NCH - _NB), _NCH):
        scat[i % _NB].wait()


@functools.cache
def _make_gather():
    return functools.partial(
        pl.kernel,
        mesh=plsc.VectorSubcoreMesh(core_axis_name="c", subcore_axis_name="s"),
        compiler_params=pltpu.CompilerParams(
            use_tc_tiling_on_sc=False, needs_layout_passes=False),
        out_type=jax.ShapeDtypeStruct((_NPIX, _E), jnp.float32),
        scratch_types=[
            pltpu.VMEM((_IDXW,), jnp.int32),
            pltpu.VMEM((_BPW,), jnp.int32),
            [pltpu.VMEM((_CH, _E), jnp.float32) for _ in range(_NB)],
            [pltpu.SemaphoreType.DMA for _ in range(_NB)],
            [pltpu.SemaphoreType.DMA for _ in range(_NB)],
        ],
    )(_gather_body)


def _proj_body_first(emb_ref, w2_ref, b_ref, out_ref):
    w2 = w2_ref[...]
    bias = b_ref[...]
    for i in range(_KP):
        e2 = emb_ref[pl.ds(i * _W, _W), :]
        r2 = lax.dot_general(w2, e2, (((1,), (1,)), ((), ())),
                             preferred_element_type=jnp.float32)
        out_ref[0, :, 2 * i, :] = r2[:_OC] + bias
        out_ref[0, :, 2 * i + 1, :] = r2[_OC:] + bias


def _proj_body(prev_ref, emb_ref, w2_ref, b_ref, out_ref):
    del prev_ref
    _proj_body_first(emb_ref, w2_ref, b_ref, out_ref)


def _project(prev, emb_b, w2, bias2d, b):
    emb2 = emb_b.reshape(_NPIX // 2, 2 * _E)
    common = dict(
        grid=(_NK,),
        out_specs=pl.BlockSpec((1, _OC, 2 * _KP, _W), lambda k, b=b: (b, 0, k, 0)),
        out_shape=jax.ShapeDtypeStruct((_B, _OC, _H, _W), jnp.float32),
    )
    emb_spec = pl.BlockSpec((_KP * _W, 2 * _E), lambda k: (k, 0))
    w_spec = pl.BlockSpec((2 * _OC, 2 * _E), lambda k: (0, 0))
    b_spec = pl.BlockSpec((_OC, 1), lambda k: (0, 0))
    if prev is None:
        return pl.pallas_call(
            _proj_body_first,
            in_specs=[emb_spec, w_spec, b_spec],
            **common,
        )(emb2, w2, bias2d)
    return pl.pallas_call(
        _proj_body,
        in_specs=[pl.BlockSpec(memory_space=pl.ANY),
                  emb_spec, w_spec, b_spec],
        input_output_aliases={0: 0},
        **common,
    )(prev, emb2, w2, bias2d)


def kernel(x, embedding_table, proj_weight, proj_bias):
    x_r = x.reshape(_B * _C, _H, _W)
    bias2d = proj_bias.reshape(_OC, 1)
    z = jnp.zeros((_OC, _E), proj_weight.dtype)
    w2 = jnp.concatenate(
        [jnp.concatenate([proj_weight, z], axis=1),
         jnp.concatenate([z, proj_weight], axis=1)], axis=0)
    gather = _make_gather()
    out = None
    for b in range(_B):
        acc = _hash_sum(x_r, b)
        idx = (acc.astype(jnp.float32) / jnp.float32(_C)).astype(jnp.int32)
        emb_b = gather(embedding_table, idx.reshape(_NPIX))
        out = _project(out, emb_b, w2, bias2d, b)
    return out
